# baseline (speedup 1.0000x reference)
import jax
import jax.numpy as jnp
from jax import lax
from jax.experimental import pallas as pl
from jax.experimental.pallas import tpu as pltpu
from jax.experimental.pallas import tpu_sc as plsc

VOCAB = 100000
EMBED_DIM = 128
BATCH = 4096
HIST = 50

ROW_BLK = 4096
NUM_BLK = (VOCAB + ROW_BLK - 1) // ROW_BLK
S_LEN = NUM_BLK * ROW_BLK
GROUPS = ROW_BLK // EMBED_DIM

NUM_WORKERS = 32
ROWS_PER_W = BATCH // NUM_WORKERS
CHUNK = ROWS_PER_W * HIST
VECS = CHUNK // 16
PERIOD = 400 // 16


def _scores_body(tab_ref, w_ref, b_ref, out_ref):
    for r in range(GROUPS):
        seg = tab_ref[pl.ds(r * EMBED_DIM, EMBED_DIM), :]
        out_ref[pl.ds(r, 1), :] = (
            jax.lax.dot_general(
                w_ref[...], seg, (((1,), (1,)), ((), ())),
                preferred_element_type=jnp.float32,
            )
            + b_ref[0]
        )


def _scores(emb_table, w, b):
    return pl.pallas_call(
        _scores_body,
        grid=(NUM_BLK,),
        in_specs=[
            pl.BlockSpec((ROW_BLK, EMBED_DIM), lambda i: (i, 0)),
            pl.BlockSpec((1, EMBED_DIM), lambda i: (0, 0)),
            pl.BlockSpec((1,), lambda i: (0,)),
        ],
        out_specs=pl.BlockSpec((GROUPS, EMBED_DIM), lambda i: (i, 0)),
        out_shape=jax.ShapeDtypeStruct((NUM_BLK * GROUPS, EMBED_DIM), jnp.float32),
    )(emb_table, w, b)


def _pool_body(s_hbm, sent_hbm, out_hbm, s_v, sent_v, a0, a1, a2, a3, sem):
    wid = lax.axis_index("s") * 2 + lax.axis_index("c")
    s_copy = pltpu.async_copy(s_hbm, s_v, sem)
    pltpu.sync_copy(sent_hbm.at[pl.ds(wid * ROWS_PER_W, ROWS_PER_W), :], sent_v)
    banks = (a0, a1, a2, a3)
    for a in banks:
        for c in range(4):
            a[pl.ds(c * 16, 16)] = jnp.zeros((16,), jnp.float32)
    s_copy.wait()

    def rc_vecs(u):
        start = (u * 16) % HIST
        rstart = (u * 16) // HIST
        raw = start + lax.iota(jnp.int32, 16)
        wrap = raw >= HIST
        col = jnp.where(wrap, raw - HIST, raw)
        r_off = rstart + jnp.where(wrap, 1, 0)
        return r_off, col

    def half_group(j, u0, u1):
        jrow = j * 8
        idxs = []
        for u in range(u0, u1):
            r_off, col = rc_vecs(u)
            idxs.append(plsc.load_gather(sent_v, [jrow + r_off, col]))
        words = [
            plsc.load_gather(s_v, [lax.shift_right_logical(ix, 1)])
            for ix in idxs
        ]
        for k, u in enumerate(range(u0, u1)):
            _, col = rc_vecs(u)
            w = words[k]
            hi = lax.bitwise_and(w, jnp.int32(-65536))
            lo = lax.shift_left(w, 16)
            odd = lax.bitwise_and(idxs[k], 1) == 1
            vals = plsc.bitcast(jnp.where(odd, hi, lo), jnp.float32)
            plsc.addupdate_scatter(banks[u % 4], [col], vals)

    def body(j, carry):
        half_group(j, 0, PERIOD // 2)
        half_group(j, PERIOD // 2, PERIOD)
        return carry

    lax.fori_loop(0, VECS // PERIOD, body, 0)
    for c in range(4):
        d = pl.ds(c * 16, 16)
        a0[d] = a0[d] + a1[d] + a2[d] + a3[d]
    pltpu.sync_copy(a0, out_hbm.at[wid])


_pool = pl.kernel(
    _pool_body,
    out_type=jax.ShapeDtypeStruct((NUM_WORKERS, 64), jnp.float32),
    mesh=plsc.VectorSubcoreMesh(core_axis_name="c", subcore_axis_name="s"),
    scratch_types=[
        pltpu.VMEM((S_LEN // 2,), jnp.int32),
        pltpu.VMEM((ROWS_PER_W, HIST), jnp.int32),
        pltpu.VMEM((64,), jnp.float32),
        pltpu.VMEM((64,), jnp.float32),
        pltpu.VMEM((64,), jnp.float32),
        pltpu.VMEM((64,), jnp.float32),
        pltpu.SemaphoreType.DMA,
    ],
    compiler_params=pltpu.CompilerParams(needs_layout_passes=False),
)


def _finalize_body(p_ref, out_ref):
    tot = jnp.sum(p_ref[...], axis=0) * (1.0 / BATCH)
    out_ref[...] = jax.nn.sigmoid(tot)


def _finalize(partials):
    return pl.pallas_call(
        _finalize_body,
        out_shape=jax.ShapeDtypeStruct((64,), jnp.float32),
    )(partials)


def kernel(sentence, emb_table, W, b):
    s = _scores(emb_table, W, b).reshape(S_LEN)
    s_pack = jax.lax.bitcast_convert_type(
        s.astype(jnp.bfloat16).reshape(S_LEN // 2, 2), jnp.int32
    )
    partials = _pool(s_pack, sentence.astype(jnp.int32))
    out64 = _finalize(partials)
    return out64[:HIST].reshape(HIST, 1)


# speedup vs baseline: 1.8440x; 1.8440x over previous
import jax
import jax.numpy as jnp
from jax import lax
from jax.experimental import pallas as pl
from jax.experimental.pallas import tpu as pltpu
from jax.experimental.pallas import tpu_sc as plsc

VOCAB = 100000
EMBED_DIM = 128
BATCH = 4096
HIST = 50

ROW_BLK = 8192
NUM_BLK = (VOCAB + ROW_BLK - 1) // ROW_BLK
S_LEN = NUM_BLK * ROW_BLK
GROUPS = ROW_BLK // EMBED_DIM
HALF_G = GROUPS // 2

NUM_WORKERS = 32
ROWS_PER_W = BATCH // NUM_WORKERS
CHUNK = ROWS_PER_W * HIST
VECS = CHUNK // 16
PERIOD = 400 // 16


def _rne_bf16_bits(x):
    bits = jax.lax.bitcast_convert_type(x, jnp.int32)
    rnd = bits + 0x7FFF + lax.bitwise_and(lax.shift_right_logical(bits, 16), 1)
    return lax.shift_right_logical(rnd, 16)


def _scores_body(tab_ref, w_ref, b_ref, out_ref):
    def group_scores(g):
        seg = tab_ref[pl.ds(g * EMBED_DIM, EMBED_DIM), :]
        return (
            jax.lax.dot_general(
                w_ref[...], seg, (((1,), (1,)), ((), ())),
                preferred_element_type=jnp.float32,
            )
            + b_ref[0]
        )

    for g in range(HALF_G):
        lo = _rne_bf16_bits(group_scores(g))
        hi = _rne_bf16_bits(group_scores(g + HALF_G))
        out_ref[pl.ds(g, 1), :] = lax.bitwise_or(lo, lax.shift_left(hi, 16))


def _scores(emb_table, w, b):
    return pl.pallas_call(
        _scores_body,
        grid=(NUM_BLK,),
        in_specs=[
            pl.BlockSpec((ROW_BLK, EMBED_DIM), lambda i: (i, 0)),
            pl.BlockSpec((1, EMBED_DIM), lambda i: (0, 0)),
            pl.BlockSpec((1,), lambda i: (0,)),
        ],
        out_specs=pl.BlockSpec((HALF_G, EMBED_DIM), lambda i: (i, 0)),
        out_shape=jax.ShapeDtypeStruct((NUM_BLK * HALF_G, EMBED_DIM), jnp.int32),
    )(emb_table, w, b)


def _pool_body(s_hbm, sent_hbm, out_hbm, s_v, sent_v, a0, a1, a2, a3, sem):
    wid = lax.axis_index("s") * 2 + lax.axis_index("c")
    s_copy = pltpu.async_copy(s_hbm, s_v, sem)
    pltpu.sync_copy(sent_hbm.at[pl.ds(wid * ROWS_PER_W, ROWS_PER_W), :], sent_v)
    banks = (a0, a1, a2, a3)
    for a in banks:
        for c in range(4):
            a[pl.ds(c * 16, 16)] = jnp.zeros((16,), jnp.float32)
    s_copy.wait()

    def rc_vecs(u):
        start = (u * 16) % HIST
        rstart = (u * 16) // HIST
        raw = start + lax.iota(jnp.int32, 16)
        wrap = raw >= HIST
        col = jnp.where(wrap, raw - HIST, raw)
        r_off = rstart + jnp.where(wrap, 1, 0)
        return r_off, col

    def half_group(j, u0, u1):
        jrow = j * 8
        idxs = []
        for u in range(u0, u1):
            r_off, col = rc_vecs(u)
            idxs.append(plsc.load_gather(sent_v, [jrow + r_off, col]))
        words = [
            plsc.load_gather(
                s_v,
                [lax.bitwise_or(
                    lax.shift_left(lax.shift_right_logical(ix, 13), 12),
                    lax.bitwise_and(ix, 4095),
                )],
            )
            for ix in idxs
        ]
        for k, u in enumerate(range(u0, u1)):
            _, col = rc_vecs(u)
            w = words[k]
            hi = lax.bitwise_and(w, jnp.int32(-65536))
            lo = lax.shift_left(w, 16)
            is_hi = lax.bitwise_and(lax.shift_right_logical(idxs[k], 12), 1) == 1
            vals = plsc.bitcast(jnp.where(is_hi, hi, lo), jnp.float32)
            plsc.addupdate_scatter(banks[u % 4], [col], vals)

    def body(j, carry):
        half_group(j, 0, PERIOD // 2)
        half_group(j, PERIOD // 2, PERIOD)
        return carry

    lax.fori_loop(0, VECS // PERIOD, body, 0)
    for c in range(4):
        d = pl.ds(c * 16, 16)
        a0[d] = a0[d] + a1[d] + a2[d] + a3[d]
    pltpu.sync_copy(a0, out_hbm.at[wid])


_pool = pl.kernel(
    _pool_body,
    out_type=jax.ShapeDtypeStruct((NUM_WORKERS, 64), jnp.float32),
    mesh=plsc.VectorSubcoreMesh(core_axis_name="c", subcore_axis_name="s"),
    scratch_types=[
        pltpu.VMEM((S_LEN // 2,), jnp.int32),
        pltpu.VMEM((ROWS_PER_W, HIST), jnp.int32),
        pltpu.VMEM((64,), jnp.float32),
        pltpu.VMEM((64,), jnp.float32),
        pltpu.VMEM((64,), jnp.float32),
        pltpu.VMEM((64,), jnp.float32),
        pltpu.SemaphoreType.DMA,
    ],
    compiler_params=pltpu.CompilerParams(needs_layout_passes=False),
)


def _finalize_body(p_ref, out_ref):
    tot = jnp.sum(p_ref[...], axis=0) * (1.0 / BATCH)
    out_ref[...] = jax.nn.sigmoid(tot)


def _finalize(partials):
    return pl.pallas_call(
        _finalize_body,
        out_shape=jax.ShapeDtypeStruct((64,), jnp.float32),
    )(partials)


def kernel(sentence, emb_table, W, b):
    s_pack = _scores(emb_table, W, b).reshape(S_LEN // 2)
    partials = _pool(s_pack, sentence.astype(jnp.int32))
    out64 = _finalize(partials)
    return out64[:HIST].reshape(HIST, 1)


# speedup vs baseline: 1.8745x; 1.0166x over previous
import jax
import jax.numpy as jnp
from jax import lax
from jax.experimental import pallas as pl
from jax.experimental.pallas import tpu as pltpu
from jax.experimental.pallas import tpu_sc as plsc

VOCAB = 100000
EMBED_DIM = 128
BATCH = 4096
HIST = 50

ROW_BLK = 16384
NUM_BLK = (VOCAB + ROW_BLK - 1) // ROW_BLK
S_LEN = NUM_BLK * ROW_BLK
GROUPS = ROW_BLK // EMBED_DIM
HALF_G = GROUPS // 2
BLK_SHIFT = ROW_BLK.bit_length() - 1
HALF_BLK = ROW_BLK // 2

NUM_WORKERS = 32
ROWS_PER_W = BATCH // NUM_WORKERS
CHUNK = ROWS_PER_W * HIST
VECS = CHUNK // 16
PERIOD = 400 // 16


def _rne_bf16_bits(x):
    bits = jax.lax.bitcast_convert_type(x, jnp.int32)
    rnd = bits + 0x7FFF + lax.bitwise_and(lax.shift_right_logical(bits, 16), 1)
    return lax.shift_right_logical(rnd, 16)


def _scores_body(tab_ref, w_ref, b_ref, out_ref):
    def group_scores(g):
        seg = tab_ref[pl.ds(g * EMBED_DIM, EMBED_DIM), :]
        return (
            jax.lax.dot_general(
                w_ref[...], seg, (((1,), (1,)), ((), ())),
                preferred_element_type=jnp.float32,
            )
            + b_ref[0]
        )

    for g in range(HALF_G):
        lo = _rne_bf16_bits(group_scores(g))
        hi = _rne_bf16_bits(group_scores(g + HALF_G))
        out_ref[pl.ds(g, 1), :] = lax.bitwise_or(lo, lax.shift_left(hi, 16))


def _scores(emb_table, w, b):
    return pl.pallas_call(
        _scores_body,
        grid=(NUM_BLK,),
        in_specs=[
            pl.BlockSpec((ROW_BLK, EMBED_DIM), lambda i: (i, 0)),
            pl.BlockSpec((1, EMBED_DIM), lambda i: (0, 0)),
            pl.BlockSpec((1,), lambda i: (0,)),
        ],
        out_specs=pl.BlockSpec((HALF_G, EMBED_DIM), lambda i: (i, 0)),
        out_shape=jax.ShapeDtypeStruct((NUM_BLK * HALF_G, EMBED_DIM), jnp.int32),
    )(emb_table, w, b)


def _pool_body(s_hbm, sent_hbm, out_hbm, s_v, sent_v, a0, a1, a2, a3, sem):
    wid = lax.axis_index("s") * 2 + lax.axis_index("c")
    s_copy = pltpu.async_copy(s_hbm, s_v, sem)
    pltpu.sync_copy(sent_hbm.at[pl.ds(wid * ROWS_PER_W, ROWS_PER_W), :], sent_v)
    banks = (a0, a1, a2, a3)
    for a in banks:
        for c in range(4):
            a[pl.ds(c * 16, 16)] = jnp.zeros((16,), jnp.float32)
    s_copy.wait()

    def rc_vecs(u):
        start = (u * 16) % HIST
        rstart = (u * 16) // HIST
        raw = start + lax.iota(jnp.int32, 16)
        wrap = raw >= HIST
        col = jnp.where(wrap, raw - HIST, raw)
        r_off = rstart + jnp.where(wrap, 1, 0)
        return r_off, col

    def half_group(j, u0, u1):
        jrow = j * 8
        idxs = []
        for u in range(u0, u1):
            r_off, col = rc_vecs(u)
            idxs.append(plsc.load_gather(sent_v, [jrow + r_off, col]))
        words = [
            plsc.load_gather(
                s_v,
                [lax.bitwise_or(
                    lax.shift_left(
                        lax.shift_right_logical(ix, BLK_SHIFT), BLK_SHIFT - 1
                    ),
                    lax.bitwise_and(ix, HALF_BLK - 1),
                )],
            )
            for ix in idxs
        ]
        for k, u in enumerate(range(u0, u1)):
            _, col = rc_vecs(u)
            w = words[k]
            hi = lax.bitwise_and(w, jnp.int32(-65536))
            lo = lax.shift_left(w, 16)
            is_hi = (
                lax.bitwise_and(lax.shift_right_logical(idxs[k], BLK_SHIFT - 1), 1)
                == 1
            )
            vals = plsc.bitcast(jnp.where(is_hi, hi, lo), jnp.float32)
            plsc.addupdate_scatter(banks[u % 4], [col], vals)

    def body(j, carry):
        half_group(j, 0, PERIOD // 2)
        half_group(j, PERIOD // 2, PERIOD)
        return carry

    lax.fori_loop(0, VECS // PERIOD, body, 0)
    for c in range(4):
        d = pl.ds(c * 16, 16)
        a0[d] = a0[d] + a1[d] + a2[d] + a3[d]
    pltpu.sync_copy(a0, out_hbm.at[wid])


_pool = pl.kernel(
    _pool_body,
    out_type=jax.ShapeDtypeStruct((NUM_WORKERS, 64), jnp.float32),
    mesh=plsc.VectorSubcoreMesh(core_axis_name="c", subcore_axis_name="s"),
    scratch_types=[
        pltpu.VMEM((S_LEN // 2,), jnp.int32),
        pltpu.VMEM((ROWS_PER_W, HIST), jnp.int32),
        pltpu.VMEM((64,), jnp.float32),
        pltpu.VMEM((64,), jnp.float32),
        pltpu.VMEM((64,), jnp.float32),
        pltpu.VMEM((64,), jnp.float32),
        pltpu.SemaphoreType.DMA,
    ],
    compiler_params=pltpu.CompilerParams(needs_layout_passes=False),
)


def _finalize_body(p_ref, out_ref):
    tot = jnp.sum(p_ref[...], axis=0) * (1.0 / BATCH)
    out_ref[...] = jax.nn.sigmoid(tot)


def _finalize(partials):
    return pl.pallas_call(
        _finalize_body,
        out_shape=jax.ShapeDtypeStruct((64,), jnp.float32),
    )(partials)


def kernel(sentence, emb_table, W, b):
    s_pack = _scores(emb_table, W, b).reshape(S_LEN // 2)
    partials = _pool(s_pack, sentence.astype(jnp.int32))
    out64 = _finalize(partials)
    return out64[:HIST].reshape(HIST, 1)


# speedup vs baseline: 2.0150x; 1.0749x over previous
import jax
import jax.numpy as jnp
from jax import lax
from jax.experimental import pallas as pl
from jax.experimental.pallas import tpu as pltpu
from jax.experimental.pallas import tpu_sc as plsc

VOCAB = 100000
EMBED_DIM = 128
BATCH = 4096
HIST = 50

ROW_BLK = 16384
NUM_BLK = (VOCAB + ROW_BLK - 1) // ROW_BLK
S_LEN = NUM_BLK * ROW_BLK
GROUPS = ROW_BLK // EMBED_DIM
HALF_G = GROUPS // 2
BLK_SHIFT = ROW_BLK.bit_length() - 1
HALF_BLK = ROW_BLK // 2

NUM_WORKERS = 32
ROWS_PER_W = BATCH // NUM_WORKERS
CHUNK = ROWS_PER_W * HIST
VECS = CHUNK // 16
PERIOD = 400 // 16


def _rne_bf16_bits(x):
    bits = jax.lax.bitcast_convert_type(x, jnp.int32)
    rnd = bits + 0x7FFF + lax.bitwise_and(lax.shift_right_logical(bits, 16), 1)
    return lax.shift_right_logical(rnd, 16)


def _scores_body(tab_ref, w_ref, b_ref, out_ref):
    def group_scores(g):
        seg = tab_ref[pl.ds(g * EMBED_DIM, EMBED_DIM), :]
        return (
            jax.lax.dot_general(
                w_ref[...], seg, (((1,), (1,)), ((), ())),
                preferred_element_type=jnp.float32,
            )
            + b_ref[0]
        )

    for g in range(HALF_G):
        lo = _rne_bf16_bits(group_scores(g))
        hi = _rne_bf16_bits(group_scores(g + HALF_G))
        out_ref[pl.ds(g, 1), :] = lax.bitwise_or(lo, lax.shift_left(hi, 16))


def _scores(emb_table, w, b):
    return pl.pallas_call(
        _scores_body,
        grid=(NUM_BLK,),
        in_specs=[
            pl.BlockSpec((ROW_BLK, EMBED_DIM), lambda i: (i, 0)),
            pl.BlockSpec((1, EMBED_DIM), lambda i: (0, 0)),
            pl.BlockSpec((1,), lambda i: (0,)),
        ],
        out_specs=pl.BlockSpec((HALF_G, EMBED_DIM), lambda i: (i, 0)),
        out_shape=jax.ShapeDtypeStruct((NUM_BLK * HALF_G, EMBED_DIM), jnp.int32),
    )(emb_table, w, b)


def _pool_body(s_hbm, sent_hbm, out_hbm, s_shr, s_v, sent_v, a0, a1, a2, a3, sem):
    sid = lax.axis_index("s")
    wid = sid * 2 + lax.axis_index("c")

    @pl.when(sid == 0)
    def _():
        pltpu.sync_copy(s_hbm, s_shr)

    pltpu.sync_copy(sent_hbm.at[pl.ds(wid * ROWS_PER_W, ROWS_PER_W), :], sent_v)
    plsc.subcore_barrier()
    s_copy = pltpu.async_copy(s_shr, s_v, sem)
    banks = (a0, a1, a2, a3)
    for a in banks:
        for c in range(4):
            a[pl.ds(c * 16, 16)] = jnp.zeros((16,), jnp.float32)
    s_copy.wait()

    def rc_vecs(u):
        start = (u * 16) % HIST
        rstart = (u * 16) // HIST
        raw = start + lax.iota(jnp.int32, 16)
        wrap = raw >= HIST
        col = jnp.where(wrap, raw - HIST, raw)
        r_off = rstart + jnp.where(wrap, 1, 0)
        return r_off, col

    def half_group(j, u0, u1):
        jrow = j * 8
        idxs = []
        for u in range(u0, u1):
            r_off, col = rc_vecs(u)
            idxs.append(plsc.load_gather(sent_v, [jrow + r_off, col]))
        words = [
            plsc.load_gather(
                s_v,
                [lax.bitwise_or(
                    lax.shift_left(
                        lax.shift_right_logical(ix, BLK_SHIFT), BLK_SHIFT - 1
                    ),
                    lax.bitwise_and(ix, HALF_BLK - 1),
                )],
            )
            for ix in idxs
        ]
        for k, u in enumerate(range(u0, u1)):
            _, col = rc_vecs(u)
            w = words[k]
            hi = lax.bitwise_and(w, jnp.int32(-65536))
            lo = lax.shift_left(w, 16)
            is_hi = (
                lax.bitwise_and(lax.shift_right_logical(idxs[k], BLK_SHIFT - 1), 1)
                == 1
            )
            vals = plsc.bitcast(jnp.where(is_hi, hi, lo), jnp.float32)
            plsc.addupdate_scatter(banks[u % 4], [col], vals)

    def body(j, carry):
        half_group(j, 0, PERIOD // 2)
        half_group(j, PERIOD // 2, PERIOD)
        return carry

    lax.fori_loop(0, VECS // PERIOD, body, 0)
    for c in range(4):
        d = pl.ds(c * 16, 16)
        a0[d] = a0[d] + a1[d] + a2[d] + a3[d]
    pltpu.sync_copy(a0, out_hbm.at[wid])


_pool = pl.kernel(
    _pool_body,
    out_type=jax.ShapeDtypeStruct((NUM_WORKERS, 64), jnp.float32),
    mesh=plsc.VectorSubcoreMesh(core_axis_name="c", subcore_axis_name="s"),
    scratch_types=[
        pltpu.VMEM_SHARED((S_LEN // 2,), jnp.int32),
        pltpu.VMEM((S_LEN // 2,), jnp.int32),
        pltpu.VMEM((ROWS_PER_W, HIST), jnp.int32),
        pltpu.VMEM((64,), jnp.float32),
        pltpu.VMEM((64,), jnp.float32),
        pltpu.VMEM((64,), jnp.float32),
        pltpu.VMEM((64,), jnp.float32),
        pltpu.SemaphoreType.DMA,
    ],
    compiler_params=pltpu.CompilerParams(needs_layout_passes=False),
)


def _finalize_body(p_ref, out_ref):
    tot = jnp.sum(p_ref[...], axis=0) * (1.0 / BATCH)
    out_ref[...] = jax.nn.sigmoid(tot)


def _finalize(partials):
    return pl.pallas_call(
        _finalize_body,
        out_shape=jax.ShapeDtypeStruct((64,), jnp.float32),
    )(partials)


def kernel(sentence, emb_table, W, b):
    s_pack = _scores(emb_table, W, b).reshape(S_LEN // 2)
    partials = _pool(s_pack, sentence.astype(jnp.int32))
    out64 = _finalize(partials)
    return out64[:HIST].reshape(HIST, 1)


# speedup vs baseline: 2.0188x; 1.0019x over previous
import jax
import jax.numpy as jnp
from jax import lax
from jax.experimental import pallas as pl
from jax.experimental.pallas import tpu as pltpu
from jax.experimental.pallas import tpu_sc as plsc

VOCAB = 100000
EMBED_DIM = 128
BATCH = 4096
HIST = 50

ROW_BLK = 16384
NUM_BLK = (VOCAB + ROW_BLK - 1) // ROW_BLK
S_LEN = NUM_BLK * ROW_BLK
GROUPS = ROW_BLK // EMBED_DIM
HALF_G = GROUPS // 2
BLK_SHIFT = ROW_BLK.bit_length() - 1
HALF_BLK = ROW_BLK // 2

NUM_WORKERS = 32
ROWS_PER_W = BATCH // NUM_WORKERS
CHUNK = ROWS_PER_W * HIST
VECS = CHUNK // 16
PERIOD = 400 // 16


def _rne_bf16_bits(x):
    bits = jax.lax.bitcast_convert_type(x, jnp.int32)
    rnd = bits + 0x7FFF + lax.bitwise_and(lax.shift_right_logical(bits, 16), 1)
    return lax.shift_right_logical(rnd, 16)


def _scores_body(tab_ref, w_ref, b_ref, out_ref):
    def group_scores(g):
        seg = tab_ref[pl.ds(g * EMBED_DIM, EMBED_DIM), :]
        return (
            jax.lax.dot_general(
                w_ref[...], seg, (((1,), (1,)), ((), ())),
                preferred_element_type=jnp.float32,
            )
            + b_ref[0]
        )

    for g in range(HALF_G):
        lo = _rne_bf16_bits(group_scores(g))
        hi = _rne_bf16_bits(group_scores(g + HALF_G))
        out_ref[pl.ds(g, 1), :] = lax.bitwise_or(lo, lax.shift_left(hi, 16))


def _scores(emb_table, w, b):
    return pl.pallas_call(
        _scores_body,
        grid=(NUM_BLK,),
        in_specs=[
            pl.BlockSpec((ROW_BLK, EMBED_DIM), lambda i: (i, 0)),
            pl.BlockSpec((1, EMBED_DIM), lambda i: (0, 0)),
            pl.BlockSpec((1,), lambda i: (0,)),
        ],
        out_specs=pl.BlockSpec((HALF_G, EMBED_DIM), lambda i: (i, 0)),
        out_shape=jax.ShapeDtypeStruct((NUM_BLK * HALF_G, EMBED_DIM), jnp.int32),
    )(emb_table, w, b)


def _pool_body(s_hbm, sent_hbm, out_hbm, s_shr, s_v, sent_v, a0, a1, a2, a3, sem):
    sid = lax.axis_index("s")
    wid = sid * 2 + lax.axis_index("c")

    @pl.when(sid == 0)
    def _():
        pltpu.sync_copy(s_hbm, s_shr)

    pltpu.sync_copy(sent_hbm.at[pl.ds(wid * ROWS_PER_W, ROWS_PER_W), :], sent_v)
    plsc.subcore_barrier()
    s_copy = pltpu.async_copy(s_shr, s_v, sem)
    banks = (a0, a1, a2, a3)
    for a in banks:
        for c in range(4):
            a[pl.ds(c * 16, 16)] = jnp.zeros((16,), jnp.float32)
    s_copy.wait()

    def rc_vecs(u):
        start = (u * 16) % HIST
        rstart = (u * 16) // HIST
        raw = start + lax.iota(jnp.int32, 16)
        wrap = raw >= HIST
        col = jnp.where(wrap, raw - HIST, raw)
        r_off = rstart + jnp.where(wrap, 1, 0)
        return r_off, col

    def half_group(j, u0, u1):
        jrow = j * 8
        idxs = []
        for u in range(u0, u1):
            r_off, col = rc_vecs(u)
            idxs.append(plsc.load_gather(sent_v, [jrow + r_off, col]))
        words = [
            plsc.load_gather(
                s_v,
                [lax.bitwise_or(
                    lax.shift_left(
                        lax.shift_right_logical(ix, BLK_SHIFT), BLK_SHIFT - 1
                    ),
                    lax.bitwise_and(ix, HALF_BLK - 1),
                )],
            )
            for ix in idxs
        ]
        for k, u in enumerate(range(u0, u1)):
            _, col = rc_vecs(u)
            w = words[k]
            hi = lax.bitwise_and(w, jnp.int32(-65536))
            lo = lax.shift_left(w, 16)
            is_hi = (
                lax.bitwise_and(lax.shift_right_logical(idxs[k], BLK_SHIFT - 1), 1)
                == 1
            )
            vals = plsc.bitcast(jnp.where(is_hi, hi, lo), jnp.float32)
            plsc.addupdate_scatter(banks[u % 4], [col], vals)

    def body(j, carry):
        half_group(j, 0, PERIOD // 2)
        half_group(j, PERIOD // 2, PERIOD)
        return carry

    lax.fori_loop(0, VECS // PERIOD, body, 0)
    for c in range(4):
        d = pl.ds(c * 16, 16)
        a0[d] = a0[d] + a1[d] + a2[d] + a3[d]
    pltpu.sync_copy(a0, out_hbm.at[wid])


_pool = pl.kernel(
    _pool_body,
    out_type=jax.ShapeDtypeStruct((NUM_WORKERS, 64), jnp.float32),
    mesh=plsc.VectorSubcoreMesh(core_axis_name="c", subcore_axis_name="s"),
    scratch_types=[
        pltpu.VMEM_SHARED((S_LEN // 2,), jnp.int32),
        pltpu.VMEM((S_LEN // 2,), jnp.int32),
        pltpu.VMEM((ROWS_PER_W, HIST), jnp.int32),
        pltpu.VMEM((64,), jnp.float32),
        pltpu.VMEM((64,), jnp.float32),
        pltpu.VMEM((64,), jnp.float32),
        pltpu.VMEM((64,), jnp.float32),
        pltpu.SemaphoreType.DMA,
    ],
    compiler_params=pltpu.CompilerParams(
        needs_layout_passes=False, skip_device_barrier=True
    ),
)


def _finalize_body(p_ref, out_ref):
    tot = jnp.sum(p_ref[...], axis=0) * (1.0 / BATCH)
    out_ref[...] = jax.nn.sigmoid(tot)


def _finalize(partials):
    return pl.pallas_call(
        _finalize_body,
        out_shape=jax.ShapeDtypeStruct((64,), jnp.float32),
    )(partials)


def kernel(sentence, emb_table, W, b):
    s_pack = _scores(emb_table, W, b).reshape(S_LEN // 2)
    partials = _pool(s_pack, sentence.astype(jnp.int32))
    out64 = _finalize(partials)
    return out64[:HIST].reshape(HIST, 1)


# speedup vs baseline: 2.0212x; 1.0012x over previous
import jax
import jax.numpy as jnp
from jax import lax
from jax.experimental import pallas as pl
from jax.experimental.pallas import tpu as pltpu
from jax.experimental.pallas import tpu_sc as plsc

VOCAB = 100000
EMBED_DIM = 128
BATCH = 4096
HIST = 50

ROW_BLK = 16384
NUM_BLK = (VOCAB + ROW_BLK - 1) // ROW_BLK
S_LEN = NUM_BLK * ROW_BLK
GROUPS = ROW_BLK // EMBED_DIM
HALF_G = GROUPS // 2
BLK_SHIFT = ROW_BLK.bit_length() - 1
HALF_BLK = ROW_BLK // 2

NUM_WORKERS = 32
ROWS_PER_W = BATCH // NUM_WORKERS
CHUNK = ROWS_PER_W * HIST
VECS = CHUNK // 16
PERIOD = 400 // 16


def _rne_bf16_bits(x):
    bits = jax.lax.bitcast_convert_type(x, jnp.int32)
    rnd = bits + 0x7FFF + lax.bitwise_and(lax.shift_right_logical(bits, 16), 1)
    return lax.shift_right_logical(rnd, 16)


def _scores_body(tab_ref, w_ref, b_ref, out_ref):
    def group_scores(g):
        seg = tab_ref[pl.ds(g * EMBED_DIM, EMBED_DIM), :]
        return (
            jax.lax.dot_general(
                w_ref[...], seg, (((1,), (1,)), ((), ())),
                preferred_element_type=jnp.float32,
            )
            + b_ref[0]
        )

    for g in range(HALF_G):
        lo = _rne_bf16_bits(group_scores(g))
        hi = _rne_bf16_bits(group_scores(g + HALF_G))
        out_ref[pl.ds(g, 1), :] = lax.bitwise_or(lo, lax.shift_left(hi, 16))


def _scores(emb_table, w, b):
    return pl.pallas_call(
        _scores_body,
        grid=(NUM_BLK,),
        in_specs=[
            pl.BlockSpec((ROW_BLK, EMBED_DIM), lambda i: (i, 0)),
            pl.BlockSpec((1, EMBED_DIM), lambda i: (0, 0)),
            pl.BlockSpec((1,), lambda i: (0,)),
        ],
        out_specs=pl.BlockSpec((HALF_G, EMBED_DIM), lambda i: (i, 0)),
        out_shape=jax.ShapeDtypeStruct((NUM_BLK * HALF_G, EMBED_DIM), jnp.int32),
    )(emb_table, w, b)


def _pool_body(s_hbm, sent_hbm, out_hbm, s_shr, s_v, sent_v, a0, a1, a2, a3, sem):
    sid = lax.axis_index("s")
    wid = sid * 2 + lax.axis_index("c")

    @pl.when(sid == 0)
    def _():
        pltpu.sync_copy(s_hbm, s_shr)

    pltpu.sync_copy(sent_hbm.at[pl.ds(wid * ROWS_PER_W, ROWS_PER_W), :], sent_v)
    plsc.subcore_barrier()
    s_copy = pltpu.async_copy(s_shr, s_v, sem)
    banks = (a0, a1, a2, a3)
    for a in banks:
        for c in range(4):
            a[pl.ds(c * 16, 16)] = jnp.zeros((16,), jnp.float32)
    s_copy.wait()

    def rc_vecs(u):
        start = (u * 16) % HIST
        rstart = (u * 16) // HIST
        raw = start + lax.iota(jnp.int32, 16)
        wrap = raw >= HIST
        col = jnp.where(wrap, raw - HIST, raw)
        r_off = rstart + jnp.where(wrap, 1, 0)
        return r_off, col

    def half_group(j, u0, u1):
        jrow = j * 8
        idxs = []
        for u in range(u0, u1):
            r_off, col = rc_vecs(u)
            idxs.append(plsc.load_gather(sent_v, [jrow + r_off, col]))
        words = [
            plsc.load_gather(
                s_v,
                [lax.bitwise_or(
                    lax.shift_left(
                        lax.shift_right_logical(ix, BLK_SHIFT), BLK_SHIFT - 1
                    ),
                    lax.bitwise_and(ix, HALF_BLK - 1),
                )],
            )
            for ix in idxs
        ]
        for k, u in enumerate(range(u0, u1)):
            _, col = rc_vecs(u)
            w = words[k]
            hi = lax.bitwise_and(w, jnp.int32(-65536))
            lo = lax.shift_left(w, 16)
            is_hi = (
                lax.bitwise_and(lax.shift_right_logical(idxs[k], BLK_SHIFT - 1), 1)
                == 1
            )
            vals = plsc.bitcast(jnp.where(is_hi, hi, lo), jnp.float32)
            plsc.addupdate_scatter(banks[u % 4], [col], vals)

    def body(j, carry):
        half_group(j, 0, PERIOD // 2)
        half_group(j, PERIOD // 2, PERIOD)
        return carry

    lax.fori_loop(0, VECS // PERIOD, body, 0)
    for c in range(4):
        d = pl.ds(c * 16, 16)
        a0[d] = a0[d] + a1[d] + a2[d] + a3[d]
    pltpu.sync_copy(a0, out_hbm.at[wid])


_pool = pl.kernel(
    _pool_body,
    out_type=jax.ShapeDtypeStruct((NUM_WORKERS, 64), jnp.float32),
    mesh=plsc.VectorSubcoreMesh(core_axis_name="c", subcore_axis_name="s"),
    scratch_types=[
        pltpu.VMEM_SHARED((S_LEN // 2,), jnp.int32),
        pltpu.VMEM((S_LEN // 2,), jnp.int32),
        pltpu.VMEM((ROWS_PER_W, HIST), jnp.int32),
        pltpu.VMEM((64,), jnp.float32),
        pltpu.VMEM((64,), jnp.float32),
        pltpu.VMEM((64,), jnp.float32),
        pltpu.VMEM((64,), jnp.float32),
        pltpu.SemaphoreType.DMA,
    ],
    compiler_params=pltpu.CompilerParams(needs_layout_passes=False),
)


def _finalize_body(p_ref, out_ref):
    tot = jnp.sum(p_ref[...], axis=0) * (1.0 / BATCH)
    out_ref[...] = jax.nn.sigmoid(tot)


def _finalize(partials):
    return pl.pallas_call(
        _finalize_body,
        out_shape=jax.ShapeDtypeStruct((64,), jnp.float32),
    )(partials)


def kernel(sentence, emb_table, W, b):
    s_pack = _scores(emb_table, W, b).reshape(S_LEN // 2)
    partials = _pool(s_pack, sentence.astype(jnp.int32))
    out64 = _finalize(partials)
    return out64[:HIST].reshape(HIST, 1)
